# baseline (speedup 1.0000x reference)
import functools

import jax
import jax.numpy as jnp
from jax import lax
from jax.experimental import pallas as pl
from jax.experimental.pallas import tpu as pltpu
from jax.experimental.pallas import tpu_sc as plsc

B = 4096
S = 200
E = 64
C = 2
NV = E // 16

CH = 4
G = 2 * CH


def _sc_pool(ids_hbm, mask_hbm, table_hbm, out_hbm, idx_v, mask_v, rows_v,
             pooled_v, gsem):
    nc = 2
    wid = lax.axis_index("s") * nc + lax.axis_index("c")
    b_per_w = B // 32
    base = wid * b_per_w
    rounds = b_per_w // CH

    def round_body(r, _):
        b0 = base + r * CH
        pltpu.sync_copy(ids_hbm.at[pl.ds(b0 * 2, G)], idx_v)
        pltpu.sync_copy(mask_hbm.at[pl.ds(b0 * S, CH * S)], mask_v)
        descs = []
        for g in range(G):
            descs.append(
                pltpu.async_copy(table_hbm.at[idx_v.at[g]],
                                 rows_v.at[pl.ds(g * 100, 100)], gsem))
        for d in descs:
            d.wait()
        for i in range(CH):
            def s_body(s, accs):
                m = plsc.load_gather(
                    mask_v, [jnp.full((16,), i * S, jnp.int32) + s])
                row = i * S + s
                return tuple(accs[e] + m * rows_v[row, pl.ds(e * 16, 16)]
                             for e in range(NV))
            zero = jnp.zeros((16,), jnp.float32)
            accs = lax.fori_loop(0, S, s_body, (zero,) * NV)
            for e in range(NV):
                pooled_v[i, pl.ds(e * 16, 16)] = accs[e]
        pltpu.sync_copy(pooled_v, out_hbm.at[pl.ds(b0, CH)])
        return 0

    lax.fori_loop(0, rounds, round_body, 0)


def _linear_body(x_ref, w_ref, b_ref, o_ref):
    w = w_ref[...] * (1.0 / S)
    o_ref[...] = jnp.dot(x_ref[...], w,
                         preferred_element_type=jnp.float32) + b_ref[...]


def kernel(input_ids, mask, emb_table, fc_w, fc_b):
    ids_r = input_ids.astype(jnp.int32).reshape(B * 2, S // 2)
    mask_r = mask.reshape(B * S)
    mesh = plsc.VectorSubcoreMesh(core_axis_name="c", subcore_axis_name="s")
    pooled = pl.kernel(
        _sc_pool,
        out_type=jax.ShapeDtypeStruct((B, E), jnp.float32),
        mesh=mesh,
        compiler_params=pltpu.CompilerParams(needs_layout_passes=False,
                                             use_tc_tiling_on_sc=False),
        scratch_types=[
            pltpu.VMEM((G, S // 2), jnp.int32),
            pltpu.VMEM((CH * S,), jnp.float32),
            pltpu.VMEM((CH * S, E), jnp.float32),
            pltpu.VMEM((CH, E), jnp.float32),
            pltpu.SemaphoreType.DMA,
        ],
    )(ids_r, mask_r, emb_table)

    out = pl.pallas_call(
        _linear_body,
        out_shape=jax.ShapeDtypeStruct((B, C), jnp.float32),
    )(pooled, fc_w, fc_b.reshape(1, C))
    return out


# speedup vs baseline: 2.6636x; 2.6636x over previous
import functools

import jax
import jax.numpy as jnp
from jax import lax
from jax.experimental import pallas as pl
from jax.experimental.pallas import tpu as pltpu
from jax.experimental.pallas import tpu_sc as plsc

B = 4096
S = 200
E = 64
C = 2
V = 1000000

VB = 8192
CH = 4
NW = 32
CHUNKS = ((0, 104), (104, 96))


def _proj_body(w_ref, tt_ref, p0_ref, p1_ref):
    w = w_ref[...] * (1.0 / S)
    chunk = tt_ref[...]
    p0_ref[...] = jnp.sum(chunk * w[:, 0:1], axis=0)
    p1_ref[...] = jnp.sum(chunk * w[:, 1:2], axis=0)


def _sc_pool(ids_hbm, mask_hbm, p0_hbm, p1_hbm, out_hbm,
             idx_v, mask_v, r0_v, r1_v, acc_v, gsem):
    wid = lax.axis_index("s") * 2 + lax.axis_index("c")
    b_per_w = B // NW
    base = wid * b_per_w
    rounds = b_per_w // CH
    tail_keep = (lax.iota(jnp.int32, 16) >= 8).astype(jnp.float32)

    def round_body(r, _):
        b0 = base + r * CH
        pltpu.sync_copy(ids_hbm.at[pl.ds(b0, CH)], idx_v)
        pltpu.sync_copy(mask_hbm.at[pl.ds(b0, CH)], mask_v)
        descs = []
        for i in range(CH):
            for (off, ln) in CHUNKS:
                idx = idx_v.at[i, pl.ds(off, ln)]
                descs.append(pltpu.async_copy(
                    p0_hbm.at[idx], r0_v.at[i, pl.ds(off, ln)], gsem))
                descs.append(pltpu.async_copy(
                    p1_hbm.at[idx], r1_v.at[i, pl.ds(off, ln)], gsem))
        for d in descs:
            d.wait()
        for i in range(CH):
            acc0 = jnp.zeros((16,), jnp.float32)
            acc1 = jnp.zeros((16,), jnp.float32)
            for k in range(S // 16):
                m = mask_v[i, pl.ds(k * 16, 16)]
                acc0 = acc0 + m * r0_v[i, pl.ds(k * 16, 16)]
                acc1 = acc1 + m * r1_v[i, pl.ds(k * 16, 16)]
            m = mask_v[i, pl.ds(S - 16, 16)] * tail_keep
            acc0 = acc0 + m * r0_v[i, pl.ds(S - 16, 16)]
            acc1 = acc1 + m * r1_v[i, pl.ds(S - 16, 16)]
            acc_v[i, pl.ds(0, 16)] = acc0
            acc_v[i, pl.ds(16, 16)] = acc1
        pltpu.sync_copy(acc_v, out_hbm.at[pl.ds(b0, CH)])
        return 0

    lax.fori_loop(0, rounds, round_body, 0)


def _epilogue_body(x_ref, b_ref, o_ref):
    x = x_ref[...]
    lanes = lax.broadcasted_iota(jnp.int32, (2 * 16, C), 0)
    cols = lax.broadcasted_iota(jnp.int32, (2 * 16, C), 1)
    sel = (lanes // 16 == cols).astype(jnp.float32)
    o_ref[...] = jnp.dot(x, sel, preferred_element_type=jnp.float32) + b_ref[...]


def kernel(input_ids, mask, emb_table, fc_w, fc_b):
    ids = input_ids.astype(jnp.int32)

    nblocks = pl.cdiv(V, VB)
    p0, p1 = pl.pallas_call(
        _proj_body,
        grid=(nblocks,),
        in_specs=[
            pl.BlockSpec((E, C), lambda i: (0, 0)),
            pl.BlockSpec((E, VB), lambda i: (0, i)),
        ],
        out_specs=[
            pl.BlockSpec((VB,), lambda i: (i,)),
            pl.BlockSpec((VB,), lambda i: (i,)),
        ],
        out_shape=[jax.ShapeDtypeStruct((V,), jnp.float32)] * 2,
    )(fc_w, emb_table.T)

    mesh = plsc.VectorSubcoreMesh(core_axis_name="c", subcore_axis_name="s")
    partials = pl.kernel(
        _sc_pool,
        out_type=jax.ShapeDtypeStruct((B, 2 * 16), jnp.float32),
        mesh=mesh,
        compiler_params=pltpu.CompilerParams(needs_layout_passes=False,
                                             use_tc_tiling_on_sc=False),
        scratch_types=[
            pltpu.VMEM((CH, S), jnp.int32),
            pltpu.VMEM((CH, S), jnp.float32),
            pltpu.VMEM((CH, S), jnp.float32),
            pltpu.VMEM((CH, S), jnp.float32),
            pltpu.VMEM((CH, 2 * 16), jnp.float32),
            pltpu.SemaphoreType.DMA,
        ],
    )(ids, mask, p0, p1)

    out = pl.pallas_call(
        _epilogue_body,
        out_shape=jax.ShapeDtypeStruct((B, C), jnp.float32),
    )(partials, fc_b.reshape(1, C))
    return out


# speedup vs baseline: 3.6786x; 1.3811x over previous
import functools

import jax
import jax.numpy as jnp
from jax import lax
from jax.experimental import pallas as pl
from jax.experimental.pallas import tpu as pltpu
from jax.experimental.pallas import tpu_sc as plsc

B = 4096
S = 200
E = 64
C = 2
V = 1000000

VB = 16384
CH = 4
NW = 32
BPW = B // NW
CHUNKS = ((0, 104), (104, 96))


def _proj_body(w_ref, tt_ref, p0_ref, p1_ref):
    w = w_ref[...] * (1.0 / S)
    chunk = tt_ref[...]
    res = lax.dot_general(w, chunk, (((0,), (0,)), ((), ())),
                          preferred_element_type=jnp.float32)
    p0_ref[...] = res[0]
    p1_ref[...] = res[1]


def _sc_pool(ids_hbm, mask_hbm, p0_hbm, p1_hbm, out_hbm,
             idx_v, mask_v, r0_v, r1_v, acc_v, gsem):
    wid = lax.axis_index("s") * 2 + lax.axis_index("c")
    base = wid * BPW
    rounds = BPW // CH
    tail_keep = (lax.iota(jnp.int32, 16) >= 8).astype(jnp.float32)

    pltpu.sync_copy(ids_hbm.at[pl.ds(base, BPW)], idx_v)
    pltpu.sync_copy(mask_hbm.at[pl.ds(base, BPW)], mask_v)

    def round_body(r, _):
        i0 = r * CH
        descs = []
        for i in range(CH):
            for (off, ln) in CHUNKS:
                idx = idx_v.at[i0 + i, pl.ds(off, ln)]
                descs.append(pltpu.async_copy(
                    p0_hbm.at[idx], r0_v.at[i, pl.ds(off, ln)], gsem))
                descs.append(pltpu.async_copy(
                    p1_hbm.at[idx], r1_v.at[i, pl.ds(off, ln)], gsem))
        for d in descs:
            d.wait()
        for i in range(CH):
            acc0 = jnp.zeros((16,), jnp.float32)
            acc1 = jnp.zeros((16,), jnp.float32)
            for k in range(S // 16):
                m = mask_v[i0 + i, pl.ds(k * 16, 16)]
                acc0 = acc0 + m * r0_v[i, pl.ds(k * 16, 16)]
                acc1 = acc1 + m * r1_v[i, pl.ds(k * 16, 16)]
            m = mask_v[i0 + i, pl.ds(S - 16, 16)] * tail_keep
            acc0 = acc0 + m * r0_v[i, pl.ds(S - 16, 16)]
            acc1 = acc1 + m * r1_v[i, pl.ds(S - 16, 16)]
            acc_v[i0 + i, pl.ds(0, 16)] = acc0
            acc_v[i0 + i, pl.ds(16, 16)] = acc1
        return 0

    lax.fori_loop(0, rounds, round_body, 0)
    pltpu.sync_copy(acc_v, out_hbm.at[pl.ds(base, BPW)])


def _epilogue_body(x_ref, b_ref, o_ref):
    x = x_ref[...]
    lanes = lax.broadcasted_iota(jnp.int32, (2 * 16, C), 0)
    cols = lax.broadcasted_iota(jnp.int32, (2 * 16, C), 1)
    sel = (lanes // 16 == cols).astype(jnp.float32)
    o_ref[...] = jnp.dot(x, sel, preferred_element_type=jnp.float32) + b_ref[...]


def kernel(input_ids, mask, emb_table, fc_w, fc_b):
    ids = input_ids.astype(jnp.int32)

    nblocks = pl.cdiv(V, VB)
    p0, p1 = pl.pallas_call(
        _proj_body,
        grid=(nblocks,),
        in_specs=[
            pl.BlockSpec((E, C), lambda i: (0, 0)),
            pl.BlockSpec((E, VB), lambda i: (0, i)),
        ],
        out_specs=[
            pl.BlockSpec((VB,), lambda i: (i,)),
            pl.BlockSpec((VB,), lambda i: (i,)),
        ],
        out_shape=[jax.ShapeDtypeStruct((V,), jnp.float32)] * 2,
    )(fc_w, emb_table.T)

    mesh = plsc.VectorSubcoreMesh(core_axis_name="c", subcore_axis_name="s")
    partials = pl.kernel(
        _sc_pool,
        out_type=jax.ShapeDtypeStruct((B, 2 * 16), jnp.float32),
        mesh=mesh,
        compiler_params=pltpu.CompilerParams(needs_layout_passes=False,
                                             use_tc_tiling_on_sc=False),
        scratch_types=[
            pltpu.VMEM((BPW, S), jnp.int32),
            pltpu.VMEM((BPW, S), jnp.float32),
            pltpu.VMEM((CH, S), jnp.float32),
            pltpu.VMEM((CH, S), jnp.float32),
            pltpu.VMEM((BPW, 2 * 16), jnp.float32),
            pltpu.SemaphoreType.DMA,
        ],
    )(ids, mask, p0, p1)

    out = pl.pallas_call(
        _epilogue_body,
        out_shape=jax.ShapeDtypeStruct((B, C), jnp.float32),
    )(partials, fc_b.reshape(1, C))
    return out


# speedup vs baseline: 4.6488x; 1.2638x over previous
import functools

import jax
import jax.numpy as jnp
from jax import lax
from jax.experimental import pallas as pl
from jax.experimental.pallas import tpu as pltpu
from jax.experimental.pallas import tpu_sc as plsc

B = 4096
S = 200
E = 64
C = 2
V = 1000000

VB = 16384
NW = 32
BPW = B // NW
NJ = BPW // 16


def _proj_body(w_ref, tt_ref, p0_ref, p1_ref):
    w = w_ref[...] * (1.0 / S)
    chunk = tt_ref[...]
    res = lax.dot_general(w, chunk, (((0,), (0,)), ((), ())),
                          preferred_element_type=jnp.float32)
    p0_ref[...] = res[0]
    p1_ref[...] = res[1]


def _sc_pool(ids_hbm, mask_hbm, p0_hbm, p1_hbm, out_hbm,
             idx_v, mask_v, r0_v, r1_v, out_v, gsem):
    wid = lax.axis_index("s") * 2 + lax.axis_index("c")
    base = wid * BPW

    pltpu.sync_copy(ids_hbm.at[:, pl.ds(base, BPW)], idx_v)
    pltpu.sync_copy(mask_hbm.at[:, pl.ds(base, BPW)], mask_v)

    def issue_body(s, _):
        pltpu.async_copy(p0_hbm.at[idx_v.at[s, :]], r0_v.at[s, :], gsem)
        pltpu.async_copy(p1_hbm.at[idx_v.at[s, :]], r1_v.at[s, :], gsem)
        return 0

    lax.fori_loop(0, S, issue_body, 0)

    zero = jnp.zeros((16,), jnp.float32)

    def drain_body(s, accs):
        pltpu.make_async_copy(p0_hbm.at[idx_v.at[s, :]], r0_v.at[s, :],
                              gsem).wait()
        pltpu.make_async_copy(p1_hbm.at[idx_v.at[s, :]], r1_v.at[s, :],
                              gsem).wait()
        out = []
        for j in range(NJ):
            m = mask_v[s, pl.ds(j * 16, 16)]
            out.append(accs[2 * j] + m * r0_v[s, pl.ds(j * 16, 16)])
            out.append(accs[2 * j + 1] + m * r1_v[s, pl.ds(j * 16, 16)])
        return tuple(out)

    accs = lax.fori_loop(0, S, drain_body, (zero,) * (2 * NJ))
    for j in range(NJ):
        out_v[0, pl.ds(j * 16, 16)] = accs[2 * j]
        out_v[1, pl.ds(j * 16, 16)] = accs[2 * j + 1]
    pltpu.sync_copy(out_v, out_hbm.at[:, pl.ds(base, BPW)])


def _epilogue_body(x_ref, b_ref, o_ref):
    o_ref[...] = x_ref[...] + b_ref[...]


def kernel(input_ids, mask, emb_table, fc_w, fc_b):
    ids_t = input_ids.astype(jnp.int32).T
    mask_t = mask.T

    nblocks = pl.cdiv(V, VB)
    p0, p1 = pl.pallas_call(
        _proj_body,
        grid=(nblocks,),
        in_specs=[
            pl.BlockSpec((E, C), lambda i: (0, 0)),
            pl.BlockSpec((E, VB), lambda i: (0, i)),
        ],
        out_specs=[
            pl.BlockSpec((VB,), lambda i: (i,)),
            pl.BlockSpec((VB,), lambda i: (i,)),
        ],
        out_shape=[jax.ShapeDtypeStruct((V,), jnp.float32)] * 2,
    )(fc_w, emb_table.T)

    mesh = plsc.VectorSubcoreMesh(core_axis_name="c", subcore_axis_name="s")
    pooled = pl.kernel(
        _sc_pool,
        out_type=jax.ShapeDtypeStruct((C, B), jnp.float32),
        mesh=mesh,
        compiler_params=pltpu.CompilerParams(needs_layout_passes=False,
                                             use_tc_tiling_on_sc=False),
        scratch_types=[
            pltpu.VMEM((S, BPW), jnp.int32),
            pltpu.VMEM((S, BPW), jnp.float32),
            pltpu.VMEM((S, BPW), jnp.float32),
            pltpu.VMEM((S, BPW), jnp.float32),
            pltpu.VMEM((C, BPW), jnp.float32),
            pltpu.SemaphoreType.DMA,
        ],
    )(ids_t, mask_t, p0, p1)

    out2 = pl.pallas_call(
        _epilogue_body,
        out_shape=jax.ShapeDtypeStruct((C, B), jnp.float32),
    )(pooled, fc_b.reshape(C, 1))
    return out2.T


# speedup vs baseline: 5.5614x; 1.1963x over previous
import functools

import jax
import jax.numpy as jnp
from jax import lax
from jax.experimental import pallas as pl
from jax.experimental.pallas import tpu as pltpu
from jax.experimental.pallas import tpu_sc as plsc

B = 4096
S = 200
E = 64
C = 2
V = 1000000

VB = 16384
NW = 32
BPW = B // NW
NJ = BPW // 16


def _proj_body(w_ref, tt_ref, p_ref):
    w = w_ref[...] * (1.0 / S)
    chunk = tt_ref[...]
    res = lax.dot_general(w, chunk, (((0,), (0,)), ((), ())),
                          preferred_element_type=jnp.float32)
    u0 = lax.bitcast_convert_type(res[0].astype(jnp.bfloat16),
                                  jnp.uint16).astype(jnp.uint32)
    u1 = lax.bitcast_convert_type(res[1].astype(jnp.bfloat16),
                                  jnp.uint16).astype(jnp.uint32)
    p_ref[...] = (u0 | (u1 << 16)).astype(jnp.int32)


def _sc_pool(ids_hbm, mask_hbm, p_hbm, out_hbm,
             idx_v, mask_v, r_v, out_v, gsem):
    wid = lax.axis_index("s") * 2 + lax.axis_index("c")
    base = wid * BPW

    pltpu.sync_copy(ids_hbm.at[:, pl.ds(base, BPW)], idx_v)
    pltpu.sync_copy(mask_hbm.at[:, pl.ds(base, BPW)], mask_v)

    def issue_body(s, _):
        pltpu.async_copy(p_hbm.at[idx_v.at[s, :]], r_v.at[s, :], gsem)
        return 0

    lax.fori_loop(0, S, issue_body, 0)

    zero = jnp.zeros((16,), jnp.float32)

    def drain_body(s, accs):
        pltpu.make_async_copy(p_hbm.at[idx_v.at[s, :]], r_v.at[s, :],
                              gsem).wait()
        out = []
        for j in range(NJ):
            m = mask_v[s, pl.ds(j * 16, 16)]
            pair = plsc.bitcast(r_v[s, pl.ds(j * 16, 16)], jnp.bfloat16)
            p0h, p1h = plsc.unpack(pair, format=plsc.PackFormat.INTERLEAVED)
            out.append(accs[2 * j] + m * p0h.astype(jnp.float32))
            out.append(accs[2 * j + 1] + m * p1h.astype(jnp.float32))
        return tuple(out)

    accs = lax.fori_loop(0, S, drain_body, (zero,) * (2 * NJ))
    for j in range(NJ):
        out_v[0, pl.ds(j * 16, 16)] = accs[2 * j]
        out_v[1, pl.ds(j * 16, 16)] = accs[2 * j + 1]
    pltpu.sync_copy(out_v, out_hbm.at[:, pl.ds(base, BPW)])


def _epilogue_body(x_ref, b_ref, o_ref):
    o_ref[...] = x_ref[...] + b_ref[...]


def kernel(input_ids, mask, emb_table, fc_w, fc_b):
    ids_t = input_ids.astype(jnp.int32).T
    mask_t = mask.T

    nblocks = pl.cdiv(V, VB)
    p_packed = pl.pallas_call(
        _proj_body,
        grid=(nblocks,),
        in_specs=[
            pl.BlockSpec((E, C), lambda i: (0, 0)),
            pl.BlockSpec((E, VB), lambda i: (0, i)),
        ],
        out_specs=pl.BlockSpec((VB,), lambda i: (i,)),
        out_shape=jax.ShapeDtypeStruct((V,), jnp.int32),
    )(fc_w, emb_table.T)

    mesh = plsc.VectorSubcoreMesh(core_axis_name="c", subcore_axis_name="s")
    pooled = pl.kernel(
        _sc_pool,
        out_type=jax.ShapeDtypeStruct((C, B), jnp.float32),
        mesh=mesh,
        compiler_params=pltpu.CompilerParams(needs_layout_passes=False,
                                             use_tc_tiling_on_sc=False),
        scratch_types=[
            pltpu.VMEM((S, BPW), jnp.int32),
            pltpu.VMEM((S, BPW), jnp.float32),
            pltpu.VMEM((S, BPW), jnp.int32),
            pltpu.VMEM((C, BPW), jnp.float32),
            pltpu.SemaphoreType.DMA,
        ],
    )(ids_t, mask_t, p_packed)

    out2 = pl.pallas_call(
        _epilogue_body,
        out_shape=jax.ShapeDtypeStruct((C, B), jnp.float32),
    )(pooled, fc_b.reshape(C, 1))
    return out2.T


# speedup vs baseline: 6.0122x; 1.0810x over previous
import functools

import jax
import jax.numpy as jnp
from jax import lax
from jax.experimental import pallas as pl
from jax.experimental.pallas import tpu as pltpu
from jax.experimental.pallas import tpu_sc as plsc

B = 4096
S = 200
E = 64
C = 2
V = 1000000

VB = 32768
NW = 32
BPW = B // NW
NJ = BPW // 16


def _proj_body(w_ref, tt_ref, p_ref):
    w = w_ref[...] * (1.0 / S)
    chunk = tt_ref[...]
    res = lax.dot_general(w, chunk, (((0,), (0,)), ((), ())),
                          preferred_element_type=jnp.float32)
    u0 = lax.bitcast_convert_type(res[0].astype(jnp.bfloat16),
                                  jnp.uint16).astype(jnp.uint32)
    u1 = lax.bitcast_convert_type(res[1].astype(jnp.bfloat16),
                                  jnp.uint16).astype(jnp.uint32)
    p_ref[...] = (u0 | (u1 << 16)).astype(jnp.int32)


def _sc_pool(ids_hbm, mask_hbm, p_hbm, out_hbm,
             idx_v, mask_v, r_v, out_v, gsem):
    wid = lax.axis_index("s") * 2 + lax.axis_index("c")
    base = wid * BPW

    pltpu.sync_copy(ids_hbm.at[:, pl.ds(base, BPW)], idx_v)
    pltpu.sync_copy(mask_hbm.at[:, pl.ds(base, BPW)], mask_v)

    def issue_body(s, _):
        pltpu.async_copy(p_hbm.at[idx_v.at[s, :]], r_v.at[s, :], gsem)
        return 0

    lax.fori_loop(0, S, issue_body, 0)

    zero = jnp.zeros((16,), jnp.float32)

    def drain_body(s, accs):
        pltpu.make_async_copy(p_hbm.at[idx_v.at[s, :]], r_v.at[s, :],
                              gsem).wait()
        out = []
        for j in range(NJ):
            m = mask_v[s, pl.ds(j * 16, 16)]
            pair = plsc.bitcast(r_v[s, pl.ds(j * 16, 16)], jnp.bfloat16)
            p0h, p1h = plsc.unpack(pair, format=plsc.PackFormat.INTERLEAVED)
            out.append(accs[2 * j] + m * p0h.astype(jnp.float32))
            out.append(accs[2 * j + 1] + m * p1h.astype(jnp.float32))
        return tuple(out)

    accs = lax.fori_loop(0, S, drain_body, (zero,) * (2 * NJ))
    for j in range(NJ):
        out_v[0, pl.ds(j * 16, 16)] = accs[2 * j]
        out_v[1, pl.ds(j * 16, 16)] = accs[2 * j + 1]
    pltpu.sync_copy(out_v, out_hbm.at[:, pl.ds(base, BPW)])


def _epilogue_body(x_ref, b_ref, o_ref):
    o_ref[...] = x_ref[...] + b_ref[...]


def kernel(input_ids, mask, emb_table, fc_w, fc_b):
    ids_t = input_ids.astype(jnp.int32).T
    mask_t = mask.T

    nblocks = pl.cdiv(V, VB)
    p_packed = pl.pallas_call(
        _proj_body,
        grid=(nblocks,),
        in_specs=[
            pl.BlockSpec((E, C), lambda i: (0, 0)),
            pl.BlockSpec((E, VB), lambda i: (0, i)),
        ],
        out_specs=pl.BlockSpec((VB,), lambda i: (i,)),
        out_shape=jax.ShapeDtypeStruct((V,), jnp.int32),
    )(fc_w, emb_table.T)

    mesh = plsc.VectorSubcoreMesh(core_axis_name="c", subcore_axis_name="s")
    pooled = pl.kernel(
        _sc_pool,
        out_type=jax.ShapeDtypeStruct((C, B), jnp.float32),
        mesh=mesh,
        compiler_params=pltpu.CompilerParams(needs_layout_passes=False,
                                             use_tc_tiling_on_sc=False),
        scratch_types=[
            pltpu.VMEM((S, BPW), jnp.int32),
            pltpu.VMEM((S, BPW), jnp.float32),
            pltpu.VMEM((S, BPW), jnp.int32),
            pltpu.VMEM((C, BPW), jnp.float32),
            pltpu.SemaphoreType.DMA,
        ],
    )(ids_t, mask_t, p_packed)

    out2 = pl.pallas_call(
        _epilogue_body,
        out_shape=jax.ShapeDtypeStruct((C, B), jnp.float32),
    )(pooled, fc_b.reshape(C, 1))
    return out2.T
